# Initial kernel scaffold; baseline (speedup 1.0000x reference)
import jax
import jax.numpy as jnp
from jax.experimental import pallas as pl


def kernel(x, vq_embed):
    raise NotImplementedError("write your pallas kernel here")


# speedup vs baseline: 1.1336x; 1.1336x over previous
import functools

import jax
import jax.numpy as jnp
from jax import lax
from jax.experimental import pallas as pl
from jax.experimental.pallas import tpu as pltpu
from jax.experimental.pallas import tpu_sc as plsc

_BETA = 0.25

_MB = 512
_KB = 4096


def _argmax_body(x_ref, x2_ref, emb_ref, idx_ref, bval_ref, bidx_ref):
    kb = pl.program_id(0)
    mb = pl.program_id(1)
    x = x_ref[...]
    e = emb_ref[...]

    mm = jnp.dot(x.astype(jnp.bfloat16), e.astype(jnp.bfloat16),
                 preferred_element_type=jnp.float32)
    x2 = x2_ref[...]
    e2 = jnp.sum(e * e, axis=0, keepdims=True)
    d = (x2 - 2.0 * mm) + e2
    bmin = jnp.min(d, axis=1, keepdims=True)
    bmin_r = bmin.astype(jnp.bfloat16).astype(jnp.float32)
    barg = jnp.argmin(d, axis=1).reshape(-1, 1).astype(jnp.int32) + kb * _KB

    rows = pl.ds(mb * _MB, _MB)

    @pl.when(kb == 0)
    def _():
        bval_ref[rows, :] = bmin_r
        bidx_ref[rows, :] = barg

    @pl.when(kb > 0)
    def _():
        upd = bmin < bval_ref[rows, :]
        bval_ref[rows, :] = jnp.where(upd, bmin_r, bval_ref[rows, :])
        bidx_ref[rows, :] = jnp.where(upd, barg, bidx_ref[rows, :])

    @pl.when(kb == pl.num_programs(0) - 1)
    def _():
        idx_ref[...] = bidx_ref[rows, :]


def _distances_argmax(x_flat, x2, vq_embed):
    m, c = x_flat.shape
    k = vq_embed.shape[1]
    grid = (k // _KB, m // _MB)
    return pl.pallas_call(
        _argmax_body,
        grid=grid,
        in_specs=[
            pl.BlockSpec((_MB, c), lambda kb, mb: (mb, 0)),
            pl.BlockSpec((_MB, 1), lambda kb, mb: (mb, 0)),
            pl.BlockSpec((c, _KB), lambda kb, mb: (0, kb)),
        ],
        out_specs=pl.BlockSpec((_MB, 1), lambda kb, mb: (mb, 0)),
        out_shape=jax.ShapeDtypeStruct((m, 1), jnp.int32),
        scratch_shapes=[
            pltpu.VMEM((m, 1), jnp.float32),
            pltpu.VMEM((m, 1), jnp.int32),
        ],
    )(x_flat, x2, vq_embed)


def _transpose_body(emb_ref, embt_ref):
    embt_ref[...] = emb_ref[...].astype(jnp.bfloat16).astype(jnp.float32).T


def _codebook_table(vq_embed):
    c, k = vq_embed.shape
    return pl.pallas_call(
        _transpose_body,
        grid=(k // _KB,),
        in_specs=[pl.BlockSpec((c, _KB), lambda i: (0, i))],
        out_specs=pl.BlockSpec((_KB, c), lambda i: (i, 0)),
        out_shape=jax.ShapeDtypeStruct((k, c), jnp.float32),
    )(vq_embed)


def _sc_gather(embt, idx):
    k, c = embt.shape
    (m,) = idx.shape
    nw = 32
    b_per_w = m // nw
    mesh = plsc.VectorSubcoreMesh(core_axis_name="c", subcore_axis_name="s")

    @functools.partial(
        pl.kernel,
        mesh=mesh,
        out_type=jax.ShapeDtypeStruct((m, c), jnp.float32),
        scratch_types=[
            pltpu.VMEM((b_per_w,), jnp.int32),
            pltpu.VMEM((b_per_w, c), jnp.float32),
            pltpu.SemaphoreType.DMA,
        ],
    )
    def gather_kernel(table_hbm, idx_hbm, out_hbm, idx_v, rows_v, sem):
        wid = lax.axis_index("s") * 2 + lax.axis_index("c")
        base = wid * b_per_w
        pltpu.sync_copy(idx_hbm.at[pl.ds(base, b_per_w)], idx_v)
        pltpu.async_copy(table_hbm.at[idx_v], rows_v, sem).wait()
        pltpu.sync_copy(rows_v, out_hbm.at[pl.ds(base, b_per_w)])

    return gather_kernel(embt, idx)


_LB = 1024


def _loss_body(x_ref, q_ref, idx_ref, qout_ref, loss_ref, perp_ref):
    j = pl.program_id(0)
    x = x_ref[...]
    q = q_ref[...]
    diff = q - x
    qout_ref[...] = x + diff
    sumsq = jnp.sum(diff * diff)

    idx = idx_ref[...]
    m = idx.shape[0]
    cnt = jnp.zeros((1, _LB), jnp.float32)
    for mi in range(m // _LB):
        blk = idx[mi * _LB:(mi + 1) * _LB, :]
        bins = j * _LB + lax.broadcasted_iota(jnp.int32, (_LB, _LB), 1)
        eq = (blk == bins).astype(jnp.float32)
        cnt = cnt + jnp.sum(eq, axis=0, keepdims=True)
    p = cnt / jnp.float32(m)
    ent = jnp.sum(p * jnp.log(p + 1e-10))

    sumsq2d = sumsq.reshape(1, 1)
    ent2d = ent.reshape(1, 1)

    @pl.when(j == 0)
    def _():
        loss_ref[...] = sumsq2d
        perp_ref[...] = ent2d

    @pl.when(j > 0)
    def _():
        loss_ref[...] = loss_ref[...] + sumsq2d
        perp_ref[...] = perp_ref[...] + ent2d

    @pl.when(j == pl.num_programs(0) - 1)
    def _():
        n = jnp.float32(x.shape[1]) * jnp.float32(m)
        mean = loss_ref[...] / n
        loss_ref[...] = mean + _BETA * mean
        perp_ref[...] = jnp.exp(-perp_ref[...])


def _loss_and_output(x_flat, q, idx2d):
    m, c = x_flat.shape
    grid = (m // _LB,)
    return pl.pallas_call(
        _loss_body,
        grid=grid,
        in_specs=[
            pl.BlockSpec((_LB, c), lambda j: (j, 0)),
            pl.BlockSpec((_LB, c), lambda j: (j, 0)),
            pl.BlockSpec((m, 1), lambda j: (0, 0)),
        ],
        out_specs=[
            pl.BlockSpec((_LB, c), lambda j: (j, 0)),
            pl.BlockSpec((1, 1), lambda j: (0, 0)),
            pl.BlockSpec((1, 1), lambda j: (0, 0)),
        ],
        out_shape=[
            jax.ShapeDtypeStruct((m, c), jnp.float32),
            jax.ShapeDtypeStruct((1, 1), jnp.float32),
            jax.ShapeDtypeStruct((1, 1), jnp.float32),
        ],
    )(x_flat, q, idx2d)


def kernel(x, vq_embed):
    c = x.shape[-1]
    x_flat = x.reshape(-1, c)
    x2 = (x_flat ** 2).sum(axis=1, keepdims=True)
    idx2d = _distances_argmax(x_flat, x2, vq_embed)
    embt = _codebook_table(vq_embed)
    q = _sc_gather(embt, idx2d.reshape(-1))
    quantized, loss, perp = _loss_and_output(x_flat, q, idx2d)
    return quantized.reshape(x.shape), loss.reshape(()), perp.reshape(())


# speedup vs baseline: 1.1508x; 1.0152x over previous
import functools

import jax
import jax.numpy as jnp
from jax import lax
from jax.experimental import pallas as pl
from jax.experimental.pallas import tpu as pltpu
from jax.experimental.pallas import tpu_sc as plsc

_BETA = 0.25

_MB = 1024
_KB = 4096


def _argmax_body(x_ref, x2_ref, emb_ref, idx_ref, bval_ref, bidx_ref):
    kb = pl.program_id(0)
    mb = pl.program_id(1)
    x = x_ref[...]
    e = emb_ref[...]

    mm = jnp.dot(x.astype(jnp.bfloat16), e.astype(jnp.bfloat16),
                 preferred_element_type=jnp.float32)
    x2 = x2_ref[...]
    e2 = jnp.sum(e * e, axis=0, keepdims=True)
    d = (x2 - 2.0 * mm) + e2
    bmin = jnp.min(d, axis=1, keepdims=True)
    bmin_r = bmin.astype(jnp.bfloat16).astype(jnp.float32)
    barg = jnp.argmin(d, axis=1).reshape(-1, 1).astype(jnp.int32) + kb * _KB

    rows = pl.ds(mb * _MB, _MB)

    @pl.when(kb == 0)
    def _():
        bval_ref[rows, :] = bmin_r
        bidx_ref[rows, :] = barg

    @pl.when(kb > 0)
    def _():
        upd = bmin < bval_ref[rows, :]
        bval_ref[rows, :] = jnp.where(upd, bmin_r, bval_ref[rows, :])
        bidx_ref[rows, :] = jnp.where(upd, barg, bidx_ref[rows, :])

    @pl.when(kb == pl.num_programs(0) - 1)
    def _():
        idx_ref[...] = bidx_ref[rows, :]


def _distances_argmax(x_flat, x2, vq_embed):
    m, c = x_flat.shape
    k = vq_embed.shape[1]
    grid = (k // _KB, m // _MB)
    return pl.pallas_call(
        _argmax_body,
        grid=grid,
        in_specs=[
            pl.BlockSpec((_MB, c), lambda kb, mb: (mb, 0)),
            pl.BlockSpec((_MB, 1), lambda kb, mb: (mb, 0)),
            pl.BlockSpec((c, _KB), lambda kb, mb: (0, kb)),
        ],
        out_specs=pl.BlockSpec((_MB, 1), lambda kb, mb: (mb, 0)),
        out_shape=jax.ShapeDtypeStruct((m, 1), jnp.int32),
        scratch_shapes=[
            pltpu.VMEM((m, 1), jnp.float32),
            pltpu.VMEM((m, 1), jnp.int32),
        ],
    )(x_flat, x2, vq_embed)


def _transpose_body(emb_ref, embt_ref):
    embt_ref[...] = emb_ref[...].astype(jnp.bfloat16).astype(jnp.float32).T


def _codebook_table(vq_embed):
    c, k = vq_embed.shape
    return pl.pallas_call(
        _transpose_body,
        grid=(k // _KB,),
        in_specs=[pl.BlockSpec((c, _KB), lambda i: (0, i))],
        out_specs=pl.BlockSpec((_KB, c), lambda i: (i, 0)),
        out_shape=jax.ShapeDtypeStruct((k, c), jnp.float32),
    )(vq_embed)


def _sc_gather(embt, idx):
    k, c = embt.shape
    (m,) = idx.shape
    nw = 32
    b_per_w = m // nw
    mesh = plsc.VectorSubcoreMesh(core_axis_name="c", subcore_axis_name="s")

    @functools.partial(
        pl.kernel,
        mesh=mesh,
        out_type=jax.ShapeDtypeStruct((m, c), jnp.float32),
        scratch_types=[
            pltpu.VMEM((b_per_w,), jnp.int32),
            pltpu.VMEM((b_per_w, c), jnp.float32),
            pltpu.SemaphoreType.DMA,
        ],
    )
    def gather_kernel(table_hbm, idx_hbm, out_hbm, idx_v, rows_v, sem):
        wid = lax.axis_index("s") * 2 + lax.axis_index("c")
        base = wid * b_per_w
        pltpu.sync_copy(idx_hbm.at[pl.ds(base, b_per_w)], idx_v)
        pltpu.async_copy(table_hbm.at[idx_v], rows_v, sem).wait()
        pltpu.sync_copy(rows_v, out_hbm.at[pl.ds(base, b_per_w)])

    return gather_kernel(embt, idx)


_LB = 1024


def _loss_body(x_ref, q_ref, idx_ref, qout_ref, loss_ref, perp_ref):
    j = pl.program_id(0)
    x = x_ref[...]
    q = q_ref[...]
    diff = q - x
    qout_ref[...] = x + diff
    sumsq = jnp.sum(diff * diff)

    idx = idx_ref[...]
    m = idx.shape[0]
    cnt = jnp.zeros((1, _LB), jnp.float32)
    for mi in range(m // _LB):
        blk = idx[mi * _LB:(mi + 1) * _LB, :]
        bins = j * _LB + lax.broadcasted_iota(jnp.int32, (_LB, _LB), 1)
        eq = (blk == bins).astype(jnp.float32)
        cnt = cnt + jnp.sum(eq, axis=0, keepdims=True)
    p = cnt / jnp.float32(m)
    ent = jnp.sum(p * jnp.log(p + 1e-10))

    sumsq2d = sumsq.reshape(1, 1)
    ent2d = ent.reshape(1, 1)

    @pl.when(j == 0)
    def _():
        loss_ref[...] = sumsq2d
        perp_ref[...] = ent2d

    @pl.when(j > 0)
    def _():
        loss_ref[...] = loss_ref[...] + sumsq2d
        perp_ref[...] = perp_ref[...] + ent2d

    @pl.when(j == pl.num_programs(0) - 1)
    def _():
        n = jnp.float32(x.shape[1]) * jnp.float32(m)
        mean = loss_ref[...] / n
        loss_ref[...] = mean + _BETA * mean
        perp_ref[...] = jnp.exp(-perp_ref[...])


def _loss_and_output(x_flat, q, idx2d):
    m, c = x_flat.shape
    grid = (m // _LB,)
    return pl.pallas_call(
        _loss_body,
        grid=grid,
        in_specs=[
            pl.BlockSpec((_LB, c), lambda j: (j, 0)),
            pl.BlockSpec((_LB, c), lambda j: (j, 0)),
            pl.BlockSpec((m, 1), lambda j: (0, 0)),
        ],
        out_specs=[
            pl.BlockSpec((_LB, c), lambda j: (j, 0)),
            pl.BlockSpec((1, 1), lambda j: (0, 0)),
            pl.BlockSpec((1, 1), lambda j: (0, 0)),
        ],
        out_shape=[
            jax.ShapeDtypeStruct((m, c), jnp.float32),
            jax.ShapeDtypeStruct((1, 1), jnp.float32),
            jax.ShapeDtypeStruct((1, 1), jnp.float32),
        ],
    )(x_flat, q, idx2d)


def kernel(x, vq_embed):
    c = x.shape[-1]
    x_flat = x.reshape(-1, c)
    x2 = (x_flat ** 2).sum(axis=1, keepdims=True)
    idx2d = _distances_argmax(x_flat, x2, vq_embed)
    embt = _codebook_table(vq_embed)
    q = _sc_gather(embt, idx2d.reshape(-1))
    quantized, loss, perp = _loss_and_output(x_flat, q, idx2d)
    return quantized.reshape(x.shape), loss.reshape(()), perp.reshape(())
